# baseline (speedup 1.0000x reference)
import functools

import jax
import jax.numpy as jnp
from jax import lax
from jax.experimental import pallas as pl
from jax.experimental.pallas import tpu as pltpu
from jax.experimental.pallas import tpu_sc as plsc

N_ENT = 10000
EMB = 256
BS = 128
TOPK = 100
NEG = 32
MARGIN = 1.0

NW = 32
B_PER_W = BS // NW
I_PER_W = B_PER_W * TOPK


def _rowsum_body(x_ref, o_ref):
    o_ref[...] = jnp.sum(x_ref[...], axis=1, keepdims=True)


def _row_sums(simi):
    rb = 200
    return pl.pallas_call(
        _rowsum_body,
        grid=(N_ENT // rb,),
        in_specs=[pl.BlockSpec((rb, N_ENT), lambda i: (i, 0))],
        out_specs=pl.BlockSpec((rb, 1), lambda i: (i, 0)),
        out_shape=jax.ShapeDtypeStruct((N_ENT, 1), jnp.float32),
    )(simi)


def _sc_gather(emb, idx2d, rowsum):
    mesh = plsc.VectorSubcoreMesh(core_axis_name="c", subcore_axis_name="s")

    @functools.partial(
        pl.kernel, mesh=mesh,
        out_type=[jax.ShapeDtypeStruct((NW, B_PER_W, TOPK, EMB), jnp.float32),
                  jax.ShapeDtypeStruct((BS, TOPK), jnp.float32)],
        scratch_types=[pltpu.VMEM((TOPK,), jnp.int32),
                       pltpu.VMEM((TOPK,), jnp.int32),
                       pltpu.VMEM((TOPK,), jnp.int32),
                       pltpu.VMEM((TOPK,), jnp.int32),
                       pltpu.VMEM((TOPK,), jnp.float32),
                       pltpu.VMEM((TOPK,), jnp.float32),
                       pltpu.VMEM((TOPK,), jnp.float32),
                       pltpu.VMEM((TOPK,), jnp.float32),
                       pltpu.VMEM((B_PER_W, TOPK, EMB), jnp.float32),
                       pltpu.SemaphoreType.DMA],
    )
    def body(emb_hbm, idx2d_hbm, rowsum_hbm, rows_out, simi_out,
             idx0_v, idx1_v, idx2_v, idx3_v,
             sim0_v, sim1_v, sim2_v, sim3_v, rows_v, sem):
        wid = lax.axis_index("s") * 2 + lax.axis_index("c")
        idx_refs = [idx0_v, idx1_v, idx2_v, idx3_v]
        sim_refs = [sim0_v, sim1_v, sim2_v, sim3_v]
        for b in range(B_PER_W):
            pltpu.sync_copy(idx2d_hbm.at[wid * B_PER_W + b], idx_refs[b])
        cps = [pltpu.async_copy(emb_hbm.at[idx_refs[b]], rows_v.at[b], sem)
               for b in range(B_PER_W)]
        cps += [pltpu.async_copy(rowsum_hbm.at[idx_refs[b]], sim_refs[b], sem)
                for b in range(B_PER_W)]
        for cp in cps:
            cp.wait()
        pltpu.sync_copy(rows_v, rows_out.at[wid])
        for b in range(B_PER_W):
            pltpu.sync_copy(sim_refs[b], simi_out.at[wid * B_PER_W + b])

    return body(emb, idx2d, rowsum)


def _final_body(rows_ref, simi_ref, st_ref, ro_ref, ps_ref, pr_ref,
                ns_ref, nr_ref, w_ref, b_ref, o_ref):
    emb = rows_ref[...]
    s1 = jnp.sum(emb, axis=1)
    s2 = jnp.sum(emb * emb, axis=1)
    var = (s2 - s1 * s1 * (1.0 / TOPK)) * (1.0 / (TOPK - 1))
    std = jnp.sqrt(jnp.maximum(var, 0.0))
    st = st_ref[...]
    ro = ro_ref[...]
    simi = simi_ref[...] * (1.0 / N_ENT)
    w = w_ref[...]
    z = (jnp.sum(std * w[:, :EMB], axis=1, keepdims=True)
         + jnp.sum(simi * w[:, EMB:EMB + TOPK], axis=1, keepdims=True)
         + jnp.sum(jnp.abs(ro - st) * w[:, EMB + TOPK:EMB + 2 * TOPK],
                   axis=1, keepdims=True)
         + jnp.sum((st + ro) * w[:, EMB + 2 * TOPK:EMB + 3 * TOPK],
                   axis=1, keepdims=True)
         + jnp.sum(st * w[:, EMB + 3 * TOPK:EMB + 4 * TOPK],
                   axis=1, keepdims=True)
         + jnp.sum(ro * w[:, EMB + 4 * TOPK:EMB + 5 * TOPK],
                   axis=1, keepdims=True)
         + b_ref[...])
    alpha = jax.nn.sigmoid(z)
    ps = ps_ref[...].reshape(BS, 1)
    pr = pr_ref[...].reshape(BS, 1)
    pos = alpha * ps + (1.0 - alpha) * pr
    neg = alpha * ns_ref[...] + (1.0 - alpha) * nr_ref[...]
    hinge = jnp.maximum(MARGIN - pos + neg, 0.0)
    o_ref[...] = (jnp.sum(hinge) * (1.0 / (BS * NEG))).reshape(1, 1)


def _finalize(rows, simi_sums, st, ro, ps, pr, ns, nr, w, b, interpret=False):
    return pl.pallas_call(
        _final_body,
        out_shape=jax.ShapeDtypeStruct((1, 1), jnp.float32),
        interpret=interpret,
    )(rows, simi_sums, st, ro, ps, pr, ns, nr, w, b)


def kernel(pos_stelp_score, pos_rotate_score, ent_idx, neg_stelp_scores,
           neg_rotate_scores, stelp_scores, rotate_scores, stelp_ent_emb,
           simi_score_mtx, proj_w, proj_b):
    idx2d = ent_idx.astype(jnp.int32).reshape(BS, TOPK)
    rowsum = _row_sums(simi_score_mtx)
    rows, simi_sums = _sc_gather(stelp_ent_emb, idx2d, rowsum.reshape(N_ENT))
    loss = _finalize(rows.reshape(BS, TOPK, EMB), simi_sums,
                     stelp_scores, rotate_scores,
                     pos_stelp_score, pos_rotate_score,
                     neg_stelp_scores, neg_rotate_scores,
                     proj_w, proj_b.reshape(1, 1))
    return loss.reshape(())


# speedup vs baseline: 1.0361x; 1.0361x over previous
import functools

import jax
import jax.numpy as jnp
from jax import lax
from jax.experimental import pallas as pl
from jax.experimental.pallas import tpu as pltpu
from jax.experimental.pallas import tpu_sc as plsc

N_ENT = 10000
EMB = 256
BS = 128
TOPK = 100
NEG = 32
MARGIN = 1.0

NW = 32
B_PER_W = BS // NW
I_PER_W = B_PER_W * TOPK


def _rowsum_body(x_ref, o_ref):
    o_ref[...] = jnp.sum(x_ref[...], axis=1, keepdims=True)


def _row_sums(simi):
    rb = 200
    return pl.pallas_call(
        _rowsum_body,
        grid=(N_ENT // rb,),
        in_specs=[pl.BlockSpec((rb, N_ENT), lambda i: (i, 0))],
        out_specs=pl.BlockSpec((rb, 1), lambda i: (i, 0)),
        out_shape=jax.ShapeDtypeStruct((N_ENT, 1), jnp.float32),
    )(simi)


def _sc_gather(emb, idx2d, rowsum):
    mesh = plsc.VectorSubcoreMesh(core_axis_name="c", subcore_axis_name="s")
    L = 16
    DC = EMB // L
    DH = DC // 2
    KU = 2
    KPAD = 104

    @functools.partial(
        pl.kernel, mesh=mesh,
        out_type=[jax.ShapeDtypeStruct((BS, EMB), jnp.float32),
                  jax.ShapeDtypeStruct((BS, EMB), jnp.float32),
                  jax.ShapeDtypeStruct((BS, KPAD), jnp.float32)],
        scratch_types=[pltpu.VMEM((KPAD,), jnp.int32),
                       pltpu.VMEM((KPAD,), jnp.int32),
                       pltpu.VMEM((KPAD,), jnp.int32),
                       pltpu.VMEM((KPAD,), jnp.int32),
                       pltpu.VMEM((KPAD,), jnp.float32),
                       pltpu.VMEM((KPAD,), jnp.float32),
                       pltpu.VMEM((KPAD,), jnp.float32),
                       pltpu.VMEM((KPAD,), jnp.float32),
                       pltpu.VMEM((B_PER_W, KPAD, EMB), jnp.float32),
                       pltpu.VMEM((B_PER_W, EMB), jnp.float32),
                       pltpu.VMEM((B_PER_W, EMB), jnp.float32),
                       pltpu.SemaphoreType.DMA,
                       pltpu.SemaphoreType.DMA,
                       pltpu.SemaphoreType.DMA,
                       pltpu.SemaphoreType.DMA,
                       pltpu.SemaphoreType.DMA],
    )
    def body(emb_hbm, idx2d_hbm, rowsum_hbm, s1_out, s2_out, simi_out,
             idx0_v, idx1_v, idx2_v, idx3_v,
             sim0_v, sim1_v, sim2_v, sim3_v, rows_v, s1_v, s2_v,
             rsem0, rsem1, rsem2, rsem3, ssem):
        wid = lax.axis_index("s") * 2 + lax.axis_index("c")
        idx_refs = [idx0_v, idx1_v, idx2_v, idx3_v]
        sim_refs = [sim0_v, sim1_v, sim2_v, sim3_v]
        rsems = [rsem0, rsem1, rsem2, rsem3]
        for b in range(B_PER_W):
            pltpu.sync_copy(idx2d_hbm.at[wid * B_PER_W + b], idx_refs[b])
        rcps = [pltpu.async_copy(emb_hbm.at[idx_refs[b]], rows_v.at[b],
                                 rsems[b]) for b in range(B_PER_W)]
        scps = [pltpu.async_copy(rowsum_hbm.at[idx_refs[b]], sim_refs[b],
                                 ssem) for b in range(B_PER_W)]
        for b in range(B_PER_W):
            rcps[b].wait()
            for half in range(2):
                base = half * DH * L

                def kbody(kk, carry):
                    out = list(carry)
                    for u in range(KU):
                        k = kk * KU + u
                        for d in range(DH):
                            v = rows_v[b, k, pl.ds(base + d * L, L)]
                            out[d] = out[d] + v
                            out[DH + d] = out[DH + d] + v * v
                    return tuple(out)

                zero = jnp.zeros((L,), jnp.float32)
                acc = lax.fori_loop(0, TOPK // KU, kbody, (zero,) * (2 * DH))
                for d in range(DH):
                    s1_v[b, pl.ds(base + d * L, L)] = acc[d]
                    s2_v[b, pl.ds(base + d * L, L)] = acc[DH + d]
        for cp in scps:
            cp.wait()
        pltpu.sync_copy(s1_v, s1_out.at[pl.ds(wid * B_PER_W, B_PER_W)])
        pltpu.sync_copy(s2_v, s2_out.at[pl.ds(wid * B_PER_W, B_PER_W)])
        for b in range(B_PER_W):
            pltpu.sync_copy(sim_refs[b], simi_out.at[wid * B_PER_W + b])

    return body(emb, idx2d, rowsum)


def _final_body(s1_ref, s2_ref, simi_ref, st_ref, ro_ref, ps_ref, pr_ref,
                ns_ref, nr_ref, w_ref, b_ref, o_ref):
    s1 = s1_ref[...]
    s2 = s2_ref[...]
    var = (s2 - s1 * s1 * (1.0 / TOPK)) * (1.0 / (TOPK - 1))
    std = jnp.sqrt(jnp.maximum(var, 0.0))
    st = st_ref[...]
    ro = ro_ref[...]
    simi = simi_ref[...][:, :TOPK] * (1.0 / N_ENT)
    w = w_ref[...]
    z = (jnp.sum(std * w[:, :EMB], axis=1, keepdims=True)
         + jnp.sum(simi * w[:, EMB:EMB + TOPK], axis=1, keepdims=True)
         + jnp.sum(jnp.abs(ro - st) * w[:, EMB + TOPK:EMB + 2 * TOPK],
                   axis=1, keepdims=True)
         + jnp.sum((st + ro) * w[:, EMB + 2 * TOPK:EMB + 3 * TOPK],
                   axis=1, keepdims=True)
         + jnp.sum(st * w[:, EMB + 3 * TOPK:EMB + 4 * TOPK],
                   axis=1, keepdims=True)
         + jnp.sum(ro * w[:, EMB + 4 * TOPK:EMB + 5 * TOPK],
                   axis=1, keepdims=True)
         + b_ref[...])
    alpha = jax.nn.sigmoid(z)
    ps = ps_ref[...].reshape(BS, 1)
    pr = pr_ref[...].reshape(BS, 1)
    pos = alpha * ps + (1.0 - alpha) * pr
    neg = alpha * ns_ref[...] + (1.0 - alpha) * nr_ref[...]
    hinge = jnp.maximum(MARGIN - pos + neg, 0.0)
    o_ref[...] = (jnp.sum(hinge) * (1.0 / (BS * NEG))).reshape(1, 1)


def _finalize(s1, s2, simi_sums, st, ro, ps, pr, ns, nr, w, b,
              interpret=False):
    return pl.pallas_call(
        _final_body,
        out_shape=jax.ShapeDtypeStruct((1, 1), jnp.float32),
        interpret=interpret,
    )(s1, s2, simi_sums, st, ro, ps, pr, ns, nr, w, b)


def kernel(pos_stelp_score, pos_rotate_score, ent_idx, neg_stelp_scores,
           neg_rotate_scores, stelp_scores, rotate_scores, stelp_ent_emb,
           simi_score_mtx, proj_w, proj_b):
    idx2d = ent_idx.astype(jnp.int32).reshape(BS, TOPK)
    idx2d = jnp.concatenate([idx2d, idx2d[:, TOPK - 4:]], axis=1)
    rowsum = _row_sums(simi_score_mtx)
    s1, s2, simi_sums = _sc_gather(stelp_ent_emb, idx2d, rowsum.reshape(N_ENT))
    loss = _finalize(s1, s2, simi_sums,
                     stelp_scores, rotate_scores,
                     pos_stelp_score, pos_rotate_score,
                     neg_stelp_scores, neg_rotate_scores,
                     proj_w, proj_b.reshape(1, 1))
    return loss.reshape(())


# speedup vs baseline: 1.0436x; 1.0072x over previous
import functools

import jax
import jax.numpy as jnp
from jax import lax
from jax.experimental import pallas as pl
from jax.experimental.pallas import tpu as pltpu
from jax.experimental.pallas import tpu_sc as plsc

N_ENT = 10000
EMB = 256
BS = 128
TOPK = 100
NEG = 32
MARGIN = 1.0

NW = 32
B_PER_W = BS // NW
I_PER_W = B_PER_W * TOPK


def _rowsum_body(x_ref, o_ref):
    o_ref[...] = jnp.sum(x_ref[...], axis=1, keepdims=True)


def _row_sums(simi):
    rb = 200
    return pl.pallas_call(
        _rowsum_body,
        grid=(N_ENT // rb,),
        in_specs=[pl.BlockSpec((rb, N_ENT), lambda i: (i, 0))],
        out_specs=pl.BlockSpec((rb, 1), lambda i: (i, 0)),
        out_shape=jax.ShapeDtypeStruct((N_ENT, 1), jnp.float32),
    )(simi)


def _sc_gather(emb, idx2d, rowsum):
    mesh = plsc.VectorSubcoreMesh(core_axis_name="c", subcore_axis_name="s")
    L = 16
    DC = EMB // L
    DH = DC // 2
    KU = 4
    KPAD = 104

    @functools.partial(
        pl.kernel, mesh=mesh,
        out_type=[jax.ShapeDtypeStruct((BS, EMB), jnp.float32),
                  jax.ShapeDtypeStruct((BS, EMB), jnp.float32),
                  jax.ShapeDtypeStruct((BS, KPAD), jnp.float32)],
        scratch_types=[pltpu.VMEM((KPAD,), jnp.int32),
                       pltpu.VMEM((KPAD,), jnp.int32),
                       pltpu.VMEM((KPAD,), jnp.int32),
                       pltpu.VMEM((KPAD,), jnp.int32),
                       pltpu.VMEM((KPAD,), jnp.float32),
                       pltpu.VMEM((KPAD,), jnp.float32),
                       pltpu.VMEM((KPAD,), jnp.float32),
                       pltpu.VMEM((KPAD,), jnp.float32),
                       pltpu.VMEM((B_PER_W, KPAD, EMB), jnp.float32),
                       pltpu.VMEM((B_PER_W, EMB), jnp.float32),
                       pltpu.VMEM((B_PER_W, EMB), jnp.float32),
                       pltpu.SemaphoreType.DMA,
                       pltpu.SemaphoreType.DMA,
                       pltpu.SemaphoreType.DMA,
                       pltpu.SemaphoreType.DMA,
                       pltpu.SemaphoreType.DMA,
                       pltpu.SemaphoreType.DMA],
    )
    def body(emb_hbm, idx2d_hbm, rowsum_hbm, s1_out, s2_out, simi_out,
             idx0_v, idx1_v, idx2_v, idx3_v,
             sim0_v, sim1_v, sim2_v, sim3_v, rows_v, s1_v, s2_v,
             rsem0, rsem1, rsem2, rsem3, ssem, isem):
        wid = lax.axis_index("s") * 2 + lax.axis_index("c")
        idx_refs = [idx0_v, idx1_v, idx2_v, idx3_v]
        sim_refs = [sim0_v, sim1_v, sim2_v, sim3_v]
        rsems = [rsem0, rsem1, rsem2, rsem3]
        icps = [pltpu.async_copy(idx2d_hbm.at[wid * B_PER_W + b], idx_refs[b],
                                 isem) for b in range(B_PER_W)]
        for cp in icps:
            cp.wait()
        rcps = [pltpu.async_copy(emb_hbm.at[idx_refs[b]], rows_v.at[b],
                                 rsems[b]) for b in range(B_PER_W)]
        scps = [pltpu.async_copy(rowsum_hbm.at[idx_refs[b]], sim_refs[b],
                                 ssem) for b in range(B_PER_W)]
        for b in range(B_PER_W):
            rcps[b].wait()
            for half in range(2):
                base = half * DH * L

                def kbody(kk, carry):
                    out = list(carry)
                    for u in range(KU):
                        k = kk * KU + u
                        for d in range(DH):
                            v = rows_v[b, k, pl.ds(base + d * L, L)]
                            out[d] = out[d] + v
                            out[DH + d] = out[DH + d] + v * v
                    return tuple(out)

                zero = jnp.zeros((L,), jnp.float32)
                acc = lax.fori_loop(0, TOPK // KU, kbody, (zero,) * (2 * DH))
                for d in range(DH):
                    s1_v[b, pl.ds(base + d * L, L)] = acc[d]
                    s2_v[b, pl.ds(base + d * L, L)] = acc[DH + d]
        for cp in scps:
            cp.wait()
        ocps = [pltpu.async_copy(s1_v, s1_out.at[pl.ds(wid * B_PER_W,
                                                       B_PER_W)], isem),
                pltpu.async_copy(s2_v, s2_out.at[pl.ds(wid * B_PER_W,
                                                       B_PER_W)], isem)]
        ocps += [pltpu.async_copy(sim_refs[b],
                                  simi_out.at[wid * B_PER_W + b], isem)
                 for b in range(B_PER_W)]
        for cp in ocps:
            cp.wait()

    return body(emb, idx2d, rowsum)


def _final_body(s1_ref, s2_ref, simi_ref, st_ref, ro_ref, ps_ref, pr_ref,
                ns_ref, nr_ref, w_ref, b_ref, o_ref):
    s1 = s1_ref[...]
    s2 = s2_ref[...]
    var = (s2 - s1 * s1 * (1.0 / TOPK)) * (1.0 / (TOPK - 1))
    std = jnp.sqrt(jnp.maximum(var, 0.0))
    st = st_ref[...]
    ro = ro_ref[...]
    simi = simi_ref[...][:, :TOPK] * (1.0 / N_ENT)
    w = w_ref[...]
    z = (jnp.sum(std * w[:, :EMB], axis=1, keepdims=True)
         + jnp.sum(simi * w[:, EMB:EMB + TOPK], axis=1, keepdims=True)
         + jnp.sum(jnp.abs(ro - st) * w[:, EMB + TOPK:EMB + 2 * TOPK],
                   axis=1, keepdims=True)
         + jnp.sum((st + ro) * w[:, EMB + 2 * TOPK:EMB + 3 * TOPK],
                   axis=1, keepdims=True)
         + jnp.sum(st * w[:, EMB + 3 * TOPK:EMB + 4 * TOPK],
                   axis=1, keepdims=True)
         + jnp.sum(ro * w[:, EMB + 4 * TOPK:EMB + 5 * TOPK],
                   axis=1, keepdims=True)
         + b_ref[...])
    alpha = jax.nn.sigmoid(z)
    ps = ps_ref[...].reshape(BS, 1)
    pr = pr_ref[...].reshape(BS, 1)
    pos = alpha * ps + (1.0 - alpha) * pr
    neg = alpha * ns_ref[...] + (1.0 - alpha) * nr_ref[...]
    hinge = jnp.maximum(MARGIN - pos + neg, 0.0)
    o_ref[...] = (jnp.sum(hinge) * (1.0 / (BS * NEG))).reshape(1, 1)


def _finalize(s1, s2, simi_sums, st, ro, ps, pr, ns, nr, w, b,
              interpret=False):
    return pl.pallas_call(
        _final_body,
        out_shape=jax.ShapeDtypeStruct((1, 1), jnp.float32),
        interpret=interpret,
    )(s1, s2, simi_sums, st, ro, ps, pr, ns, nr, w, b)


def kernel(pos_stelp_score, pos_rotate_score, ent_idx, neg_stelp_scores,
           neg_rotate_scores, stelp_scores, rotate_scores, stelp_ent_emb,
           simi_score_mtx, proj_w, proj_b):
    idx2d = ent_idx.astype(jnp.int32).reshape(BS, TOPK)
    idx2d = jnp.concatenate([idx2d, idx2d[:, TOPK - 4:]], axis=1)
    rowsum = _row_sums(simi_score_mtx)
    s1, s2, simi_sums = _sc_gather(stelp_ent_emb, idx2d, rowsum.reshape(N_ENT))
    loss = _finalize(s1, s2, simi_sums,
                     stelp_scores, rotate_scores,
                     pos_stelp_score, pos_rotate_score,
                     neg_stelp_scores, neg_rotate_scores,
                     proj_w, proj_b.reshape(1, 1))
    return loss.reshape(())
